# baseline (speedup 1.0000x reference)
import functools

import jax
import jax.numpy as jnp
from jax import lax
from jax.experimental import pallas as pl
from jax.experimental.pallas import tpu as pltpu
from jax.experimental.pallas import tpu_sc as plsc

D = 256
NC, NS = 2, 16
NW = NC * NS
W = 64
NB = 4
R = 64
IB = 2048
CPB = IB // W


def _sc_gather(tokens_flat, table_repl):
    B = tokens_flat.shape[0]
    b_per_w = B // NW
    n_outer = b_per_w // IB
    mesh = plsc.VectorSubcoreMesh(core_axis_name="c", subcore_axis_name="s")

    @functools.partial(
        pl.kernel,
        mesh=mesh,
        out_type=jax.ShapeDtypeStruct((B, D), jnp.float32),
        scratch_types=(
            [pltpu.VMEM((IB,), jnp.int32)] * 2
            + [pltpu.VMEM((W, D), jnp.float32)] * NB
            + [pltpu.SemaphoreType.DMA] * (2 + 2 * NB)
        ),
    )
    def k(tab_hbm, idx_hbm, out_hbm, idx0, idx1, *bufs):
        rows = bufs[:NB]
        isem = bufs[NB:NB + 2]
        gs = bufs[NB + 2:NB + 2 + NB]
        ws = bufs[NB + 2 + NB:]
        idxs = (idx0, idx1)
        wid = lax.axis_index("s") * NC + lax.axis_index("c")
        base = wid * b_per_w

        def load_idx(o, p):
            pltpu.async_copy(
                idx_hbm.at[pl.ds(base + o * IB, IB)], idxs[p], isem[p]
            )

        def wait_idx(p):
            pltpu.make_async_copy(
                idx_hbm.at[pl.ds(base, IB)], idxs[p], isem[p]
            ).wait()

        def inner(o, idx_v):
            obase = base + o * IB

            @pl.loop(0, CPB, step=NB)
            def _(ci):
                handles = []
                for b in range(NB):
                    c = ci + b
                    off = obase + c * W

                    @pl.when(jnp.logical_or(o > 0, ci >= NB))
                    def _():
                        pltpu.make_async_copy(
                            rows[b], out_hbm.at[pl.ds(off, W)], ws[b]
                        ).wait()

                    handles.append(
                        pltpu.async_copy(
                            tab_hbm.at[wid * R + b * 16 + ((ci // NB) & 15)].at[
                                idx_v.at[pl.ds(c * W, W)]],
                            rows[b],
                            gs[b],
                        )
                    )
                for b in range(NB):
                    c = ci + b
                    off = obase + c * W
                    handles[b].wait()
                    pltpu.async_copy(rows[b], out_hbm.at[pl.ds(off, W)], ws[b])

        load_idx(0, 0)

        @pl.loop(0, n_outer, step=2)
        def _(o):
            load_idx(o + 1, 1)
            wait_idx(0)
            inner(o, idx0)

            @pl.when(o + 2 < n_outer)
            def _():
                load_idx(o + 2, 0)

            wait_idx(1)
            inner(o + 1, idx1)

        for b in range(NB):
            pltpu.make_async_copy(
                rows[b], out_hbm.at[pl.ds(base, W)], ws[b]
            ).wait()

    return k(table_repl, tokens_flat)


def kernel(tokens, table):
    bsz, seq = tokens.shape
    table_repl = jnp.broadcast_to(table, (NW * R,) + table.shape)
    out = _sc_gather(tokens.reshape(bsz * seq), table_repl)
    return out.reshape(bsz, seq, D)


# speedup vs baseline: 1.0036x; 1.0036x over previous
import functools

import jax
import jax.numpy as jnp
from jax import lax
from jax.experimental import pallas as pl
from jax.experimental.pallas import tpu as pltpu
from jax.experimental.pallas import tpu_sc as plsc

D = 256
NC, NS = 2, 16
NW = NC * NS
W = 64
NB = 4
R = 16
IB = 2048
CPB = IB // W


def _sc_gather(tokens_flat, table_repl):
    B = tokens_flat.shape[0]
    b_per_w = B // NW
    n_outer = b_per_w // IB
    mesh = plsc.VectorSubcoreMesh(core_axis_name="c", subcore_axis_name="s")

    @functools.partial(
        pl.kernel,
        mesh=mesh,
        out_type=jax.ShapeDtypeStruct((B, D), jnp.float32),
        scratch_types=(
            [pltpu.VMEM((IB,), jnp.int32)] * 2
            + [pltpu.VMEM((W, D), jnp.float32)] * NB
            + [pltpu.SemaphoreType.DMA] * (2 + 2 * NB)
        ),
    )
    def k(tab_hbm, idx_hbm, out_hbm, idx0, idx1, *bufs):
        rows = bufs[:NB]
        isem = bufs[NB:NB + 2]
        gs = bufs[NB + 2:NB + 2 + NB]
        ws = bufs[NB + 2 + NB:]
        idxs = (idx0, idx1)
        wid = lax.axis_index("s") * NC + lax.axis_index("c")
        base = wid * b_per_w

        def load_idx(o, p):
            pltpu.async_copy(
                idx_hbm.at[pl.ds(base + o * IB, IB)], idxs[p], isem[p]
            )

        def wait_idx(p):
            pltpu.make_async_copy(
                idx_hbm.at[pl.ds(base, IB)], idxs[p], isem[p]
            ).wait()

        def inner(o, idx_v):
            obase = base + o * IB

            @pl.loop(0, CPB, step=NB)
            def _(ci):
                handles = []
                for b in range(NB):
                    c = ci + b
                    off = obase + c * W

                    @pl.when(jnp.logical_or(o > 0, ci >= NB))
                    def _():
                        pltpu.make_async_copy(
                            rows[b], out_hbm.at[pl.ds(off, W)], ws[b]
                        ).wait()

                    handles.append(
                        pltpu.async_copy(
                            tab_hbm.at[wid * R + b * 4 + ((ci // NB) & 3)].at[
                                idx_v.at[pl.ds(c * W, W)]],
                            rows[b],
                            gs[b],
                        )
                    )
                for b in range(NB):
                    c = ci + b
                    off = obase + c * W
                    handles[b].wait()
                    pltpu.async_copy(rows[b], out_hbm.at[pl.ds(off, W)], ws[b])

        load_idx(0, 0)

        @pl.loop(0, n_outer, step=2)
        def _(o):
            load_idx(o + 1, 1)
            wait_idx(0)
            inner(o, idx0)

            @pl.when(o + 2 < n_outer)
            def _():
                load_idx(o + 2, 0)

            wait_idx(1)
            inner(o + 1, idx1)

        for b in range(NB):
            pltpu.make_async_copy(
                rows[b], out_hbm.at[pl.ds(base, W)], ws[b]
            ).wait()

    return k(table_repl, tokens_flat)


def kernel(tokens, table):
    bsz, seq = tokens.shape
    table_repl = jnp.broadcast_to(table, (NW * R,) + table.shape)
    out = _sc_gather(tokens.reshape(bsz * seq), table_repl)
    return out.reshape(bsz, seq, D)


# speedup vs baseline: 1.0040x; 1.0004x over previous
import functools

import jax
import jax.numpy as jnp
from jax import lax
from jax.experimental import pallas as pl
from jax.experimental.pallas import tpu as pltpu
from jax.experimental.pallas import tpu_sc as plsc

D = 256
NC, NS = 2, 16
NW = NC * NS
W = 64
NB = 4
R = 16
IB = 2048
CPB = IB // W


def _sc_gather(tokens_flat, table_repl):
    B = tokens_flat.shape[0]
    b_per_w = B // NW
    n_outer = b_per_w // IB
    mesh = plsc.VectorSubcoreMesh(core_axis_name="c", subcore_axis_name="s")

    @functools.partial(
        pl.kernel,
        mesh=mesh,
        out_type=jax.ShapeDtypeStruct((B, D), jnp.float32),
        scratch_types=(
            [pltpu.VMEM((IB,), jnp.int32)] * 2
            + [pltpu.VMEM((W, D), jnp.float32)] * NB
            + [pltpu.SemaphoreType.DMA] * (2 + 2 * NB)
        ),
    )
    def k(tab_hbm, idx_hbm, out_hbm, idx0, idx1, *bufs):
        rows = bufs[:NB]
        isem = bufs[NB:NB + 2]
        gs = bufs[NB + 2:NB + 2 + NB]
        ws = bufs[NB + 2 + NB:]
        idxs = (idx0, idx1)
        wid = lax.axis_index("s") * NC + lax.axis_index("c")
        base = wid * b_per_w

        def load_idx(o, p):
            pltpu.async_copy(
                idx_hbm.at[pl.ds(base + o * IB, IB)], idxs[p], isem[p]
            )

        def wait_idx(p):
            pltpu.make_async_copy(
                idx_hbm.at[pl.ds(base, IB)], idxs[p], isem[p]
            ).wait()

        def inner(o, idx_v):
            obase = base + o * IB

            @pl.loop(0, CPB, step=NB)
            def _(ci):
                handles = []
                for b in range(NB):
                    c = ci + b
                    off = obase + c * W

                    @pl.when(jnp.logical_or(o > 0, ci >= NB))
                    def _():
                        pltpu.make_async_copy(
                            rows[b], out_hbm.at[pl.ds(off, W)], ws[b]
                        ).wait()

                    handles.append(
                        pltpu.async_copy(
                            tab_hbm.at[wid * R + b * 4 + ((ci // NB) & 3)].at[
                                idx_v.at[pl.ds(c * W, W)]],
                            rows[b],
                            gs[b],
                        )
                    )
                for b in range(NB):
                    c = ci + b
                    off = obase + c * W
                    handles[b].wait()
                    pltpu.async_copy(rows[b], out_hbm.at[pl.ds(off, W)], ws[b])

        load_idx(0, 0)

        @pl.loop(0, n_outer, step=2)
        def _(o):
            load_idx(o + 1, 1)
            wait_idx(0)
            inner(o, idx0)

            @pl.when(o + 2 < n_outer)
            def _():
                load_idx(o + 2, 0)

            wait_idx(1)
            inner(o + 1, idx1)

        for b in range(NB):
            pltpu.make_async_copy(
                rows[b], out_hbm.at[pl.ds(base, W)], ws[b]
            ).wait()

    return k(table_repl, tokens_flat)


def kernel(tokens, table):
    bsz, seq = tokens.shape
    table_repl = jnp.broadcast_to(table, (NW * R,) + table.shape)
    idx = tokens.reshape(bsz * seq).astype(jnp.int32)
    out = _sc_gather(idx, table_repl)
    return out.reshape(bsz, seq, D)
